# baseline (speedup 1.0000x reference)
import jax
import jax.numpy as jnp
from jax.experimental import pallas as pl

_ROWS = 512
_COLS = 16384
_BLOCK_ROWS = 64


def _tc_body(x_ref, t_ref, o_ref):
    i = pl.program_id(0)
    x = jax.nn.sigmoid(x_ref[...])
    t = t_ref[...]
    p = jnp.sum(x * t, axis=1)
    s = jnp.sum(x, axis=1)
    ts = jnp.sum(t, axis=1)
    loss = 1.0 - (2.0 * p + 1.0) / (s + ts + 1.0)
    blk = jnp.sum(loss).reshape(1, 1)

    @pl.when(i == 0)
    def _():
        o_ref[...] = jnp.zeros((1, 1), jnp.float32)

    o_ref[...] += blk


def kernel(inputs, targets, num_boxes):
    grid = _ROWS // _BLOCK_ROWS
    out = pl.pallas_call(
        _tc_body,
        grid=(grid,),
        in_specs=[
            pl.BlockSpec((_BLOCK_ROWS, _COLS), lambda i: (i, 0)),
            pl.BlockSpec((_BLOCK_ROWS, _COLS), lambda i: (i, 0)),
        ],
        out_specs=pl.BlockSpec((1, 1), lambda i: (0, 0)),
        out_shape=jax.ShapeDtypeStruct((1, 1), jnp.float32),
    )(inputs, targets)
    return out[0, 0] / (num_boxes + 1e-06)
